# Initial kernel scaffold; baseline (speedup 1.0000x reference)
import jax
import jax.numpy as jnp
from jax.experimental import pallas as pl


def kernel(positions, table):
    raise NotImplementedError("write your pallas kernel here")


# speedup vs baseline: 3.0106x; 3.0106x over previous
import functools

import jax
import jax.numpy as jnp
from jax import lax
from jax.experimental import pallas as pl
from jax.experimental.pallas import tpu as pltpu
from jax.experimental.pallas import tpu_sc as plsc

D = 64
CHUNK = 128


def _build(N, V):
    info = plsc.get_sparse_core_info()
    NC, NS = info.num_cores, info.num_subcores
    NW = NC * NS
    assert N % NW == 0
    b_per_w = N // NW
    assert b_per_w % CHUNK == 0
    n_chunks = b_per_w // CHUNK

    mesh = plsc.VectorSubcoreMesh(core_axis_name="c", subcore_axis_name="s")

    @functools.partial(
        pl.kernel,
        mesh=mesh,
        out_type=jax.ShapeDtypeStruct((N, D), jnp.float32),
        compiler_params=pltpu.CompilerParams(use_tc_tiling_on_sc=False),
        scratch_types=[
            pltpu.VMEM((CHUNK,), jnp.int32),
            pltpu.VMEM((CHUNK, D), jnp.float32),
            pltpu.SemaphoreType.DMA,
        ],
    )
    def k(table_hbm, idx_hbm, out_hbm, idx_v, rows_v, sem):
        wid = lax.axis_index("s") * NC + lax.axis_index("c")
        base = wid * b_per_w

        def body(i, _):
            off = base + i * CHUNK
            pltpu.sync_copy(idx_hbm.at[pl.ds(off, CHUNK)], idx_v)
            pltpu.async_copy(table_hbm.at[idx_v], rows_v, sem).wait()
            pltpu.sync_copy(rows_v, out_hbm.at[pl.ds(off, CHUNK)])
            return 0

        lax.fori_loop(0, n_chunks, body, 0)

    return k


def kernel(positions, table):
    B, S = positions.shape
    V, d = table.shape
    N = B * S
    idx = positions.reshape(N).astype(jnp.int32)
    out = _build(N, V)(table, idx)
    return out.reshape(B, S, d)


# speedup vs baseline: 3.0674x; 1.0189x over previous
import functools

import jax
import jax.numpy as jnp
from jax import lax
from jax.experimental import pallas as pl
from jax.experimental.pallas import tpu as pltpu
from jax.experimental.pallas import tpu_sc as plsc

D = 64
IW = 128
K = 4
CHUNK = K * IW


def _build(N):
    info = plsc.get_sparse_core_info()
    NC, NS = info.num_cores, info.num_subcores
    NW = NC * NS
    assert N % (NW * CHUNK) == 0
    b_per_w = N // NW
    n_chunks = b_per_w // CHUNK
    assert n_chunks % 2 == 0
    G = n_chunks // 2

    mesh = plsc.VectorSubcoreMesh(core_axis_name="c", subcore_axis_name="s")

    @functools.partial(
        pl.kernel,
        mesh=mesh,
        out_type=jax.ShapeDtypeStruct((N, D), jnp.float32),
        compiler_params=pltpu.CompilerParams(use_tc_tiling_on_sc=False),
        scratch_types=[
            pltpu.VMEM((2, K, IW), jnp.int32),
            pltpu.VMEM((2, CHUNK, D), jnp.float32),
            pltpu.SemaphoreType.DMA,
            pltpu.SemaphoreType.DMA,
            pltpu.SemaphoreType.DMA,
            pltpu.SemaphoreType.DMA,
            pltpu.SemaphoreType.DMA,
        ],
    )
    def k(table_hbm, idx_hbm, out_hbm, idx_v, rows_v, gat_sem,
          idx_sem0, idx_sem1, out_sem0, out_sem1):
        wid = lax.axis_index("s") * NC + lax.axis_index("c")
        base = wid * b_per_w
        rbase = base // IW

        idx_sems = (idx_sem0, idx_sem1)
        out_sems = (out_sem0, out_sem1)

        pltpu.async_copy(idx_hbm.at[pl.ds(rbase, K)], idx_v.at[0], idx_sem0)
        pltpu.async_copy(idx_hbm.at[pl.ds(rbase + K, K)], idx_v.at[1], idx_sem1)

        def body(g, _):
            for b in (0, 1):
                i = 2 * g + b
                off = base + i * CHUNK
                pltpu.make_async_copy(
                    idx_hbm.at[pl.ds(rbase, K)], idx_v.at[b], idx_sems[b]
                ).wait()
                @pl.when(g > 0)
                def _drain_out():
                    pltpu.make_async_copy(
                        rows_v.at[b], out_hbm.at[pl.ds(base, CHUNK)], out_sems[b]
                    ).wait()
                handles = []
                for j in range(K):
                    handles.append(pltpu.async_copy(
                        table_hbm.at[idx_v.at[b, j]],
                        rows_v.at[b, pl.ds(j * IW, IW)],
                        gat_sem,
                    ))
                for h in handles:
                    h.wait()
                pltpu.async_copy(
                    rows_v.at[b], out_hbm.at[pl.ds(off, CHUNK)], out_sems[b]
                )
                @pl.when(g < G - 1)
                def _prefetch():
                    pltpu.async_copy(
                        idx_hbm.at[pl.ds(rbase + (i + 2) * K, K)],
                        idx_v.at[b],
                        idx_sems[b],
                    )
            return 0

        lax.fori_loop(0, G, body, 0)

        for b in (0, 1):
            pltpu.make_async_copy(
                rows_v.at[b], out_hbm.at[pl.ds(base, CHUNK)], out_sems[b]
            ).wait()

    return k


def kernel(positions, table):
    B, S = positions.shape
    V, d = table.shape
    N = B * S
    idx = positions.reshape(N // IW, IW).astype(jnp.int32)
    out = _build(N)(table, idx)
    return out.reshape(B, S, d)


# speedup vs baseline: 5.8210x; 1.8977x over previous
import functools

import jax
import jax.numpy as jnp
from jax import lax
from jax.experimental import pallas as pl
from jax.experimental.pallas import tpu as pltpu
from jax.experimental.pallas import tpu_sc as plsc

D = 64
IW = 128
K = 4
CHUNK = K * IW


def _build(N, V):
    info = plsc.get_sparse_core_info()
    NC, NS = info.num_cores, info.num_subcores
    NW = NC * NS
    assert N % (NW * CHUNK) == 0
    b_per_w = N // NW
    n_chunks = b_per_w // CHUNK
    assert n_chunks % 2 == 0
    G = n_chunks // 2

    mesh = plsc.VectorSubcoreMesh(core_axis_name="c", subcore_axis_name="s")

    @functools.partial(
        pl.kernel,
        mesh=mesh,
        out_type=jax.ShapeDtypeStruct((N, D), jnp.float32),
        compiler_params=pltpu.CompilerParams(use_tc_tiling_on_sc=False),
        scratch_types=[
            pltpu.VMEM((2, K, IW), jnp.int32),
            pltpu.VMEM((2, CHUNK, D), jnp.float32),
            pltpu.VMEM_SHARED((V, D), jnp.float32),
            pltpu.SemaphoreType.DMA,
            pltpu.SemaphoreType.DMA,
            pltpu.SemaphoreType.DMA,
            pltpu.SemaphoreType.DMA,
            pltpu.SemaphoreType.DMA,
        ],
    )
    def k(table_hbm, idx_hbm, out_hbm, idx_v, rows_v, table_sh, gat_sem,
          idx_sem0, idx_sem1, out_sem0, out_sem1):
        wid = lax.axis_index("s") * NC + lax.axis_index("c")
        base = wid * b_per_w
        rbase = base // IW

        @pl.when(lax.axis_index("s") == 0)
        def _stage():
            pltpu.sync_copy(table_hbm, table_sh)
        plsc.subcore_barrier()

        idx_sems = (idx_sem0, idx_sem1)
        out_sems = (out_sem0, out_sem1)

        pltpu.async_copy(idx_hbm.at[pl.ds(rbase, K)], idx_v.at[0], idx_sem0)
        pltpu.async_copy(idx_hbm.at[pl.ds(rbase + K, K)], idx_v.at[1], idx_sem1)

        def body(g, _):
            for b in (0, 1):
                i = 2 * g + b
                off = base + i * CHUNK
                pltpu.make_async_copy(
                    idx_hbm.at[pl.ds(rbase, K)], idx_v.at[b], idx_sems[b]
                ).wait()
                @pl.when(g > 0)
                def _drain_out():
                    pltpu.make_async_copy(
                        rows_v.at[b], out_hbm.at[pl.ds(base, CHUNK)], out_sems[b]
                    ).wait()
                handles = []
                for j in range(K):
                    handles.append(pltpu.async_copy(
                        table_sh.at[idx_v.at[b, j]],
                        rows_v.at[b, pl.ds(j * IW, IW)],
                        gat_sem,
                    ))
                for h in handles:
                    h.wait()
                pltpu.async_copy(
                    rows_v.at[b], out_hbm.at[pl.ds(off, CHUNK)], out_sems[b]
                )
                @pl.when(g < G - 1)
                def _prefetch():
                    pltpu.async_copy(
                        idx_hbm.at[pl.ds(rbase + (i + 2) * K, K)],
                        idx_v.at[b],
                        idx_sems[b],
                    )
            return 0

        lax.fori_loop(0, G, body, 0)

        for b in (0, 1):
            pltpu.make_async_copy(
                rows_v.at[b], out_hbm.at[pl.ds(base, CHUNK)], out_sems[b]
            ).wait()

    return k


def kernel(positions, table):
    B, S = positions.shape
    V, d = table.shape
    N = B * S
    idx = positions.reshape(N // IW, IW).astype(jnp.int32)
    out = _build(N, V)(table, idx)
    return out.reshape(B, S, d)
